# Initial kernel scaffold; baseline (speedup 1.0000x reference)
import jax
import jax.numpy as jnp
from jax.experimental import pallas as pl


def kernel(inputs, codebook):
    raise NotImplementedError("write your pallas kernel here")


# speedup vs baseline: 1.2118x; 1.2118x over previous
import jax
import jax.numpy as jnp
from jax.experimental import pallas as pl

_K = 1024
_D = 64
_CCOST = 0.25


def _vq_block(x_ref, cb_ref, quant_ref, idx_ref, loss_ref):
    x = x_ref[0]
    cb = cb_ref[...]
    csq = jnp.sum(cb * cb, axis=1, keepdims=True)
    xsq = jnp.sum(x * x, axis=0)
    prod = jax.lax.dot_general(
        cb, x, (((1,), (0,)), ((), ())),
        preferred_element_type=jnp.float32)
    score = (xsq[None, :] + csq) - 2.0 * prod
    m = jnp.min(score, axis=0)
    kiota = jax.lax.broadcasted_iota(jnp.int32, score.shape, 0)
    idx = jnp.min(jnp.where(score == m[None, :], kiota, _K), axis=0)
    enc = (kiota == idx[None, :]).astype(jnp.float32)
    quant = jax.lax.dot_general(
        cb, enc, (((0,), (0,)), ((), ())),
        preferred_element_type=jnp.float32)
    quant_ref[0] = quant
    idx_ref[0, 0] = idx
    loss_ref[0, 0] = jnp.broadcast_to(jnp.sum(m), (128,))


def kernel(inputs, codebook):
    B, C, H, W = inputs.shape
    T = H * W
    x3 = inputs.reshape(B, C, T)
    quant, idx, loss = pl.pallas_call(
        _vq_block,
        grid=(B,),
        in_specs=[
            pl.BlockSpec((1, C, T), lambda b: (b, 0, 0)),
            pl.BlockSpec((_K, _D), lambda b: (0, 0)),
        ],
        out_specs=[
            pl.BlockSpec((1, C, T), lambda b: (b, 0, 0)),
            pl.BlockSpec((1, 1, T), lambda b: (b, 0, 0)),
            pl.BlockSpec((1, 1, 128), lambda b: (b, 0, 0)),
        ],
        out_shape=[
            jax.ShapeDtypeStruct((B, C, T), jnp.float32),
            jax.ShapeDtypeStruct((B, 1, T), jnp.int32),
            jax.ShapeDtypeStruct((B, 1, 128), jnp.float32),
        ],
    )(x3, codebook)
    quantized_out = quant.reshape(B, C, H, W)
    encoding_indices = idx.reshape(B * T)
    vq_loss = (1.0 + _CCOST) * jnp.sum(loss[:, 0, 0]) / (B * C * T)
    return quantized_out, vq_loss, encoding_indices
